# Initial kernel scaffold; baseline (speedup 1.0000x reference)
import jax
import jax.numpy as jnp
from jax.experimental import pallas as pl


def kernel(x, edge_index, edge_weight, W):
    raise NotImplementedError("write your pallas kernel here")


# speedup vs baseline: 4.7291x; 4.7291x over previous
import dataclasses
import functools

import jax
import jax.numpy as jnp
from jax import lax
from jax.experimental import pallas as pl
from jax.experimental.pallas import tpu as pltpu
from jax.experimental.pallas import tpu_sc as plsc

N_NODES = 10000
N_EDGES = 320000
D = 128

E_BLK = 128
N_BLOCKS = N_EDGES // E_BLK
BLOCKS_PER_CORE = N_BLOCKS // 2
N_SUBCORES = 16
ITERS = (BLOCKS_PER_CORE + N_SUBCORES - 1) // N_SUBCORES
ROWS_MAIN = 624
ROWS_TAIL = N_NODES - N_SUBCORES * ROWS_MAIN


def _mm_body(x_ref, w_ref, h_ref):
    h_ref[...] = jnp.dot(x_ref[...], w_ref[...],
                         preferred_element_type=jnp.float32)


def _matmul(x, W):
    grid = 10
    blk = N_NODES // grid
    return pl.pallas_call(
        _mm_body,
        grid=(grid,),
        in_specs=[
            pl.BlockSpec((blk, D), lambda i: (i, 0)),
            pl.BlockSpec((D, D), lambda i: (0, 0)),
        ],
        out_specs=pl.BlockSpec((blk, D), lambda i: (i, 0)),
        out_shape=jax.ShapeDtypeStruct((N_NODES, D), jnp.float32),
    )(x, W)


def _sc_body(h_hbm, src_hbm, dst_hbm, w_hbm, out_hbm,
             src_v, dst_v, w_v, rows_v, acc, sem):
    c = lax.axis_index("c")
    t = lax.axis_index("s")

    @pl.loop(0, E_BLK)
    def _zero_rows(r):
        for j in range(D // 16):
            rows_v[r, pl.ds(16 * j, 16)] = jnp.zeros((16,), jnp.float32)

    for k, sz in ((0, 128), (128, 128), (256, 128), (384, 128), (512, 112)):
        pltpu.sync_copy(rows_v.at[pl.ds(0, sz)],
                        acc.at[pl.ds(t * ROWS_MAIN + k, sz)])

    @pl.when(t == N_SUBCORES - 1)
    def _zero_tail():
        pltpu.sync_copy(rows_v.at[pl.ds(0, ROWS_TAIL)],
                        acc.at[pl.ds(N_SUBCORES * ROWS_MAIN, ROWS_TAIL)])

    plsc.subcore_barrier()

    @pl.loop(0, ITERS)
    def _edge_iter(i):
        rel = i * N_SUBCORES + t

        @pl.when(rel < BLOCKS_PER_CORE)
        def _():
            base = (c * BLOCKS_PER_CORE + rel) * E_BLK
            pltpu.sync_copy(src_hbm.at[pl.ds(base, E_BLK)], src_v)
            pltpu.sync_copy(dst_hbm.at[pl.ds(base, E_BLK)], dst_v)
            pltpu.sync_copy(w_hbm.at[pl.ds(base, E_BLK)], w_v)
            pltpu.async_copy(h_hbm.at[src_v], rows_v, sem).wait()

            @pl.loop(0, E_BLK)
            def _scale(e):
                idx = jnp.full((16,), e, jnp.int32)
                w16 = plsc.load_gather(w_v, [idx])
                for j in range(D // 16):
                    sl = pl.ds(16 * j, 16)
                    rows_v[e, sl] = rows_v[e, sl] * w16

            pltpu.sync_copy(rows_v, acc.at[dst_v], add=True)

    plsc.subcore_barrier()

    r0 = t * ROWS_MAIN
    pltpu.sync_copy(acc.at[pl.ds(r0, ROWS_MAIN)],
                    out_hbm.at[c, pl.ds(r0, ROWS_MAIN)])

    @pl.when(t == N_SUBCORES - 1)
    def _drain_tail():
        r1 = N_SUBCORES * ROWS_MAIN
        pltpu.sync_copy(acc.at[pl.ds(r1, ROWS_TAIL)],
                        out_hbm.at[c, pl.ds(r1, ROWS_TAIL)])


def _sc_aggregate(h, src, dst, w):
    mesh = plsc.VectorSubcoreMesh(core_axis_name="c", subcore_axis_name="s")
    cp = pltpu.CompilerParams()
    if "needs_layout_passes" in pltpu.CompilerParams.__dataclass_fields__:
        cp = dataclasses.replace(cp, needs_layout_passes=False)
    kern = pl.kernel(
        _sc_body,
        out_type=jax.ShapeDtypeStruct((2, N_NODES, D), jnp.float32),
        mesh=mesh,
        scratch_types=[
            pltpu.VMEM((E_BLK,), jnp.int32),
            pltpu.VMEM((E_BLK,), jnp.int32),
            pltpu.VMEM((E_BLK,), jnp.float32),
            pltpu.VMEM((E_BLK, D), jnp.float32),
            pltpu.VMEM_SHARED((N_NODES, D), jnp.float32),
            pltpu.SemaphoreType.DMA,
        ],
        compiler_params=cp,
    )
    return kern(h, src, dst, w)


def _add_body(p_ref, o_ref):
    o_ref[...] = p_ref[0] + p_ref[1]


def _sum_partials(partials):
    grid = 10
    blk = N_NODES // grid
    return pl.pallas_call(
        _add_body,
        grid=(grid,),
        in_specs=[pl.BlockSpec((2, blk, D), lambda i: (0, i, 0))],
        out_specs=pl.BlockSpec((blk, D), lambda i: (i, 0)),
        out_shape=jax.ShapeDtypeStruct((N_NODES, D), jnp.float32),
    )(partials)


def kernel(x, edge_index, edge_weight, W):
    h = _matmul(x, W)
    partials = _sc_aggregate(h, edge_index[0], edge_index[1], edge_weight)
    return _sum_partials(partials)
